# baseline (speedup 1.0000x reference)
import functools

import jax
import jax.numpy as jnp
from jax import lax
from jax.experimental import pallas as pl
from jax.experimental.pallas import tpu as pltpu
from jax.experimental.pallas import tpu_sc as plsc

NC = 2
NS = 16
NW = NC * NS
CH = 128

_SC_PARAMS = pltpu.CompilerParams(use_tc_tiling_on_sc=False)


def _sc_degree(ep, zeros8, ones8, R, iters):
    rpt = R // NS
    mesh = plsc.VectorSubcoreMesh(core_axis_name="c", subcore_axis_name="s")
    K = 8
    rounds = iters // K

    @functools.partial(
        pl.kernel,
        out_type=jax.ShapeDtypeStruct((NC, R, 8), jnp.float32),
        mesh=mesh,
        scratch_types=[
            pltpu.VMEM((iters, CH), jnp.int32),
            pltpu.VMEM((K, CH), jnp.int32),
            pltpu.VMEM((CH, 8), jnp.float32),
            pltpu.VMEM((rpt, 8), jnp.float32),
            pltpu.VMEM_SHARED((R, 8), jnp.float32),
            pltpu.SemaphoreType.DMA,
        ],
        compiler_params=_SC_PARAMS,
    )
    def k(ep_hbm, zeros_hbm, ones_hbm, out_hbm, ep_v, idxd32, ones_v,
          chunk_v, hist, sem):
        cid = lax.axis_index("c")
        sid = lax.axis_index("s")
        wid = sid * NC + cid
        row = pl.ds(sid * rpt, rpt)
        pltpu.sync_copy(zeros_hbm.at[row], chunk_v)
        pltpu.sync_copy(chunk_v, hist.at[row])
        pltpu.sync_copy(ep_hbm.at[pl.ds(wid * iters, iters)], ep_v)
        pltpu.sync_copy(ones_hbm, ones_v)
        plsc.subcore_barrier()

        def body(g, carry):
            for b in range(K):
                for j in range(CH // 16):
                    v = ep_v[g * K + b, pl.ds(j * 16, 16)]
                    idxd32[b, pl.ds(j * 16, 16)] = v >> 16
                pltpu.async_copy(
                    ones_v, hist.at[idxd32.at[b]], sem, add=True)
            for b in range(K):
                pltpu.make_async_copy(
                    ones_v, hist.at[idxd32.at[0]], sem).wait()
            return carry

        lax.fori_loop(0, rounds, body, 0)
        plsc.subcore_barrier()
        pltpu.sync_copy(hist.at[row], chunk_v)
        pltpu.sync_copy(chunk_v, out_hbm.at[cid, row])

    return k(ep, zeros8, ones8)


def _sc_edge_pass(ep, table, zeros, R, W, shift, cpt, nb, chw=CH):
    rpt = R // NS
    mesh = plsc.VectorSubcoreMesh(core_axis_name="c", subcore_axis_name="s")
    NB = nb

    @functools.partial(
        pl.kernel,
        out_type=jax.ShapeDtypeStruct((NC, R, W), jnp.float32),
        mesh=mesh,
        scratch_types=(
            [pltpu.VMEM((cpt, chw), jnp.int32),
             pltpu.VMEM((NB, chw), jnp.int32),
             pltpu.VMEM((NB, chw), jnp.int32)]
            + [pltpu.VMEM((chw, W), jnp.float32) for _ in range(NB)]
            + [pltpu.VMEM_SHARED((R, W), jnp.float32),
               pltpu.VMEM_SHARED((R, W), jnp.float32)]
            + [pltpu.SemaphoreType.DMA for _ in range(2 * NB)]
        ),
        compiler_params=_SC_PARAMS,
    )
    def k(ep_hbm, table_hbm, zeros_hbm, out_hbm,
          ep_v, idxs32, idxd32, *bufs_and_sems):
        rows = bufs_and_sems[:NB]
        acc = bufs_and_sems[NB]
        table_sh = bufs_and_sems[NB + 1]
        semg = bufs_and_sems[NB + 2:NB + 2 + NB]
        sems = bufs_and_sems[NB + 2 + NB:]
        cid = lax.axis_index("c")
        sid = lax.axis_index("s")
        wid = sid * NC + cid
        row = pl.ds(sid * rpt, rpt)

        chunks = []
        o = 0
        while o < rpt:
            c = min(chw, rpt - o)
            chunks.append((o, c))
            o += c
        pltpu.sync_copy(table_hbm.at[row], table_sh.at[row])
        pltpu.sync_copy(zeros_hbm, rows[0])
        for (o, c) in chunks:
            pltpu.sync_copy(rows[0].at[pl.ds(0, c)],
                            acc.at[pl.ds(sid * rpt + o, c)])
        pltpu.sync_copy(ep_hbm.at[pl.ds(wid * cpt, cpt)], ep_v)
        plsc.subcore_barrier()

        def widen(b, i):
            for j in range(chw // 16):
                v = ep_v[i, pl.ds(j * 16, 16)]
                lo = v & 0xFFFF
                hi = v >> 16
                if shift:
                    lo = lo >> 1
                    hi = hi >> 1
                idxs32[b, pl.ds(j * 16, 16)] = lo
                idxd32[b, pl.ds(j * 16, 16)] = hi

        def body(g, carry):
            for b in range(NB):
                @pl.when(g > 0)
                def _drain():
                    pltpu.make_async_copy(
                        rows[b], acc.at[idxd32.at[0]], sems[b]).wait()
                widen(b, g * NB + b)
                pltpu.async_copy(
                    table_sh.at[idxs32.at[b]], rows[b], semg[b])
            for b in range(NB):
                pltpu.make_async_copy(
                    table_sh.at[idxs32.at[0]], rows[b], semg[b]).wait()
                pltpu.async_copy(
                    rows[b], acc.at[idxd32.at[b]], sems[b], add=True)
            return carry

        lax.fori_loop(0, cpt // NB, body, 0)
        for b in range(NB):
            pltpu.make_async_copy(
                rows[b], acc.at[idxd32.at[0]], sems[b]).wait()
        plsc.subcore_barrier()
        live = {}
        for z, (o, c) in enumerate(chunks):
            sl = pl.ds(sid * rpt + o, c)
            b = z % NB
            if b in live:
                pltpu.make_async_copy(
                    rows[b].at[pl.ds(0, live[b])],
                    out_hbm.at[cid, pl.ds(0, live[b])], semg[b]).wait()
            pltpu.sync_copy(acc.at[sl], rows[b].at[pl.ds(0, c)])
            pltpu.async_copy(rows[b].at[pl.ds(0, c)],
                             out_hbm.at[cid, sl], semg[b])
            live[b] = c
        for b, c in live.items():
            pltpu.make_async_copy(
                rows[b].at[pl.ds(0, c)],
                out_hbm.at[cid, pl.ds(0, c)], semg[b]).wait()

    return k(ep, table, zeros)


def _tc_prep_enc(x_pad, W, degp, B=640):
    R, D = x_pad.shape
    H = W.shape[1]

    def body(x_ref, w_ref, d0_ref, d1_ref, o_ref):
        dinv = lax.rsqrt(d0_ref[0, :, 0:1] + d1_ref[0, :, 0:1] + 1.0)
        o_ref[...] = jnp.dot(x_ref[...], w_ref[...],
                             preferred_element_type=jnp.float32) * dinv

    return pl.pallas_call(
        body,
        grid=(R // B,),
        in_specs=[
            pl.BlockSpec((B, D), lambda i: (i, 0)),
            pl.BlockSpec((D, H), lambda i: (0, 0)),
            pl.BlockSpec((1, B, 8), lambda i: (0, i, 0)),
            pl.BlockSpec((1, B, 8), lambda i: (1, i, 0)),
        ],
        out_specs=pl.BlockSpec((B, H), lambda i: (i, 0)),
        out_shape=jax.ShapeDtypeStruct((R, H), jnp.float32),
    )(x_pad, W, degp, degp)


def _tc_mid(a0v, a1v, hsv, degv, W, b, B=640):
    Rc, H2 = hsv.shape
    H = H2 // 2
    D = W.shape[1]

    def body(a0_ref, a1_ref, hs_ref, d0_ref, d1_ref, w_ref, b_ref, o_ref):
        dl = d0_ref[0, :, 0:1] + d1_ref[0, :, 0:1] + 1.0
        dr = d0_ref[0, :, 8:9] + d1_ref[0, :, 8:9] + 1.0
        sl_ = (a0_ref[0, :, :H] + a1_ref[0, :, :H] + hs_ref[:, :H])
        sr_ = (a0_ref[0, :, H:] + a1_ref[0, :, H:] + hs_ref[:, H:])
        hl = jnp.maximum(sl_ * lax.rsqrt(dl) + b_ref[...], 0.0)
        hr = jnp.maximum(sr_ * lax.rsqrt(dr) + b_ref[...], 0.0)
        xc = 0.5 * (hl + hr)
        degc = dl + dr - 1.0
        o_ref[...] = jnp.dot(xc, w_ref[...],
                             preferred_element_type=jnp.float32) * lax.rsqrt(degc)

    return pl.pallas_call(
        body,
        grid=(Rc // B,),
        in_specs=[
            pl.BlockSpec((1, B, H2), lambda i: (0, i, 0)),
            pl.BlockSpec((1, B, H2), lambda i: (1, i, 0)),
            pl.BlockSpec((B, H2), lambda i: (i, 0)),
            pl.BlockSpec((1, B, 16), lambda i: (0, i, 0)),
            pl.BlockSpec((1, B, 16), lambda i: (1, i, 0)),
            pl.BlockSpec((H, D), lambda i: (0, 0)),
            pl.BlockSpec((1, H), lambda i: (0, 0)),
        ],
        out_specs=pl.BlockSpec((B, D), lambda i: (i, 0)),
        out_shape=jax.ShapeDtypeStruct((Rc, D), jnp.float32),
    )(a0v, a1v, hsv, degv, degv, W, b)


def _tc_final(accc, hds, degv, b, B=640):
    Rc, D = hds.shape

    def body(a0_ref, a1_ref, hds_ref, d0_ref, d1_ref, b_ref, o_ref):
        dl = d0_ref[0, :, 0:1] + d1_ref[0, :, 0:1] + 1.0
        dr = d0_ref[0, :, 8:9] + d1_ref[0, :, 8:9] + 1.0
        degc = dl + dr - 1.0
        xd = ((a0_ref[0] + a1_ref[0] + hds_ref[...]) * lax.rsqrt(degc)
              + b_ref[...])
        o_ref[:, :D] = xd
        o_ref[:, D:] = xd

    return pl.pallas_call(
        body,
        grid=(Rc // B,),
        in_specs=[
            pl.BlockSpec((1, B, D), lambda i: (0, i, 0)),
            pl.BlockSpec((1, B, D), lambda i: (1, i, 0)),
            pl.BlockSpec((B, D), lambda i: (i, 0)),
            pl.BlockSpec((1, B, 16), lambda i: (0, i, 0)),
            pl.BlockSpec((1, B, 16), lambda i: (1, i, 0)),
            pl.BlockSpec((1, D), lambda i: (0, 0)),
        ],
        out_specs=pl.BlockSpec((B, 2 * D), lambda i: (i, 0)),
        out_shape=jax.ShapeDtypeStruct((Rc, 2 * D), jnp.float32),
    )(accc, accc, hds, degv, degv, b)


def kernel(x, edge_index, batch, W_enc, b_enc, W_dec, b_dec):
    N, D = x.shape
    H = W_enc.shape[1]
    E = edge_index.shape[1]
    Nc = N // 2

    Rc = ((Nc + 1 + 255) // 256) * 256
    R = 2 * Rc
    S = -(-(-(-E // CH)) // (NS * 8)) * 8
    cpt = -(-(-(-S // NC)) // 4) * 4
    iters = NS * S // NW
    C_pad = max(NS * S, NW * cpt)
    pad_e = C_pad * CH - E

    epk = edge_index[0] | (edge_index[1] << 16)
    ep = jnp.concatenate(
        [epk, jnp.full((pad_e,), N | (N << 16), jnp.int32)]).reshape(-1, CH)

    zeros8 = jnp.zeros((R, 8), jnp.float32)
    ones8 = jnp.zeros((CH, 8), jnp.float32).at[:, 0].set(1.0)
    zf = jnp.zeros((CH, H), jnp.float32)
    zc = jnp.zeros((CH // 2, D), jnp.float32)
    x_pad = jnp.concatenate([x, jnp.zeros((R - N, D), x.dtype)])

    degp = _sc_degree(ep, zeros8, ones8, R, iters)
    hs = _tc_prep_enc(x_pad, W_enc, degp)
    accf = _sc_edge_pass(ep, hs, zf, R, H, False, cpt, 4)

    degv = degp.reshape(NC, Rc, 16)
    hds = _tc_mid(accf.reshape(NC, Rc, 2 * H), accf.reshape(NC, Rc, 2 * H),
                  hs.reshape(Rc, 2 * H), degv, W_dec, b_enc.reshape(1, H))
    accc = _sc_edge_pass(ep.reshape(-1, CH // 2), hds, zc, Rc, D,
                         True, 2 * cpt, 4, CH // 2)
    outd = _tc_final(accc, hds, degv, b_dec.reshape(1, D))

    return outd[:Nc].reshape(N, D)


# speedup vs baseline: 1.0744x; 1.0744x over previous
import functools

import jax
import jax.numpy as jnp
from jax import lax
from jax.experimental import pallas as pl
from jax.experimental.pallas import tpu as pltpu
from jax.experimental.pallas import tpu_sc as plsc

NC = 2
NS = 16
NW = NC * NS
CH = 128

_SC_PARAMS = pltpu.CompilerParams(use_tc_tiling_on_sc=False)


def _sc_degree(ep, zeros8, ones8, R, iters):
    rpt = R // NS
    mesh = plsc.VectorSubcoreMesh(core_axis_name="c", subcore_axis_name="s")
    K = 8
    rounds = iters // K

    @functools.partial(
        pl.kernel,
        out_type=jax.ShapeDtypeStruct((NC, R, 8), jnp.float32),
        mesh=mesh,
        scratch_types=[
            pltpu.VMEM((iters, CH), jnp.int32),
            pltpu.VMEM((K, CH), jnp.int32),
            pltpu.VMEM((CH, 8), jnp.float32),
            pltpu.VMEM((rpt, 8), jnp.float32),
            pltpu.VMEM_SHARED((R, 8), jnp.float32),
            pltpu.SemaphoreType.DMA,
        ],
        compiler_params=_SC_PARAMS,
    )
    def k(ep_hbm, zeros_hbm, ones_hbm, out_hbm, ep_v, idxd32, ones_v,
          chunk_v, hist, sem):
        cid = lax.axis_index("c")
        sid = lax.axis_index("s")
        wid = sid * NC + cid
        row = pl.ds(sid * rpt, rpt)
        pltpu.sync_copy(zeros_hbm.at[row], chunk_v)
        pltpu.sync_copy(chunk_v, hist.at[row])
        pltpu.sync_copy(ep_hbm.at[pl.ds(wid * iters, iters)], ep_v)
        pltpu.sync_copy(ones_hbm, ones_v)
        plsc.subcore_barrier()

        def body(g, carry):
            for b in range(K):
                for j in range(CH // 16):
                    v = ep_v[g * K + b, pl.ds(j * 16, 16)]
                    idxd32[b, pl.ds(j * 16, 16)] = v >> 16
                pltpu.async_copy(
                    ones_v, hist.at[idxd32.at[b]], sem, add=True)
            for b in range(K):
                pltpu.make_async_copy(
                    ones_v, hist.at[idxd32.at[0]], sem).wait()
            return carry

        lax.fori_loop(0, rounds, body, 0)
        plsc.subcore_barrier()
        pltpu.sync_copy(hist.at[row], chunk_v)
        pltpu.sync_copy(chunk_v, out_hbm.at[cid, row])

    return k(ep, zeros8, ones8)


def _sc_edge_pass(ep, table, zeros, R, W, shift, cpt, nb):
    rpt = R // NS
    mesh = plsc.VectorSubcoreMesh(core_axis_name="c", subcore_axis_name="s")
    NB = nb

    @functools.partial(
        pl.kernel,
        out_type=jax.ShapeDtypeStruct((NC, R, W), jnp.float32),
        mesh=mesh,
        scratch_types=(
            [pltpu.VMEM((cpt, CH), jnp.int32),
             pltpu.VMEM((NB, CH), jnp.int32),
             pltpu.VMEM((NB, CH), jnp.int32)]
            + [pltpu.VMEM((CH, W), jnp.float32) for _ in range(NB)]
            + [pltpu.VMEM_SHARED((R, W), jnp.float32),
               pltpu.VMEM_SHARED((R, W), jnp.float32)]
            + [pltpu.SemaphoreType.DMA for _ in range(2 * NB)]
        ),
        compiler_params=_SC_PARAMS,
    )
    def k(ep_hbm, table_hbm, zeros_hbm, out_hbm,
          ep_v, idxs32, idxd32, *bufs_and_sems):
        rows = bufs_and_sems[:NB]
        acc = bufs_and_sems[NB]
        table_sh = bufs_and_sems[NB + 1]
        semg = bufs_and_sems[NB + 2:NB + 2 + NB]
        sems = bufs_and_sems[NB + 2 + NB:]
        cid = lax.axis_index("c")
        sid = lax.axis_index("s")
        wid = sid * NC + cid
        row = pl.ds(sid * rpt, rpt)

        chunks = []
        o = 0
        while o < rpt:
            c = min(CH, rpt - o)
            chunks.append((o, c))
            o += c
        pltpu.sync_copy(table_hbm.at[row], table_sh.at[row])
        pltpu.sync_copy(zeros_hbm, rows[0])
        for (o, c) in chunks:
            pltpu.sync_copy(rows[0].at[pl.ds(0, c)],
                            acc.at[pl.ds(sid * rpt + o, c)])
        pltpu.sync_copy(ep_hbm.at[pl.ds(wid * cpt, cpt)], ep_v)
        plsc.subcore_barrier()

        def widen(b, i):
            for j in range(CH // 16):
                v = ep_v[i, pl.ds(j * 16, 16)]
                lo = v & 0xFFFF
                hi = v >> 16
                if shift:
                    lo = lo >> 1
                    hi = hi >> 1
                idxs32[b, pl.ds(j * 16, 16)] = lo
                idxd32[b, pl.ds(j * 16, 16)] = hi

        def body(g, carry):
            for b in range(NB):
                @pl.when(g > 0)
                def _drain():
                    pltpu.make_async_copy(
                        rows[b], acc.at[idxd32.at[0]], sems[b]).wait()
                widen(b, g * NB + b)
                pltpu.async_copy(
                    table_sh.at[idxs32.at[b]], rows[b], semg[b])
            for b in range(NB):
                pltpu.make_async_copy(
                    table_sh.at[idxs32.at[0]], rows[b], semg[b]).wait()
                pltpu.async_copy(
                    rows[b], acc.at[idxd32.at[b]], sems[b], add=True)
            return carry

        lax.fori_loop(0, cpt // NB, body, 0)
        for b in range(NB):
            pltpu.make_async_copy(
                rows[b], acc.at[idxd32.at[0]], sems[b]).wait()
        plsc.subcore_barrier()
        live = {}
        for z, (o, c) in enumerate(chunks):
            sl = pl.ds(sid * rpt + o, c)
            b = z % NB
            if b in live:
                pltpu.make_async_copy(
                    rows[b].at[pl.ds(0, live[b])],
                    out_hbm.at[cid, pl.ds(0, live[b])], semg[b]).wait()
            pltpu.sync_copy(acc.at[sl], rows[b].at[pl.ds(0, c)])
            pltpu.async_copy(rows[b].at[pl.ds(0, c)],
                             out_hbm.at[cid, sl], semg[b])
            live[b] = c
        for b, c in live.items():
            pltpu.make_async_copy(
                rows[b].at[pl.ds(0, c)],
                out_hbm.at[cid, pl.ds(0, c)], semg[b]).wait()

    return k(ep, table, zeros)


def _tc_prep_enc(x_pad, W, degp, B=640):
    R, D = x_pad.shape
    H = W.shape[1]

    def body(x_ref, w_ref, d0_ref, d1_ref, o_ref):
        dinv = lax.rsqrt(d0_ref[0, :, 0:1] + d1_ref[0, :, 0:1] + 1.0)
        o_ref[...] = jnp.dot(x_ref[...], w_ref[...],
                             preferred_element_type=jnp.float32) * dinv

    return pl.pallas_call(
        body,
        grid=(R // B,),
        in_specs=[
            pl.BlockSpec((B, D), lambda i: (i, 0)),
            pl.BlockSpec((D, H), lambda i: (0, 0)),
            pl.BlockSpec((1, B, 8), lambda i: (0, i, 0)),
            pl.BlockSpec((1, B, 8), lambda i: (1, i, 0)),
        ],
        out_specs=pl.BlockSpec((B, H), lambda i: (i, 0)),
        out_shape=jax.ShapeDtypeStruct((R, H), jnp.float32),
    )(x_pad, W, degp, degp)


def _tc_mid(a0v, a1v, hsv, degv, W, b, B=640):
    Rc, H2 = hsv.shape
    H = H2 // 2
    D = W.shape[1]

    def body(a0_ref, a1_ref, hs_ref, d0_ref, d1_ref, w_ref, b_ref, o_ref):
        dl = d0_ref[0, :, 0:1] + d1_ref[0, :, 0:1] + 1.0
        dr = d0_ref[0, :, 8:9] + d1_ref[0, :, 8:9] + 1.0
        sl_ = (a0_ref[0, :, :H] + a1_ref[0, :, :H] + hs_ref[:, :H])
        sr_ = (a0_ref[0, :, H:] + a1_ref[0, :, H:] + hs_ref[:, H:])
        hl = jnp.maximum(sl_ * lax.rsqrt(dl) + b_ref[...], 0.0)
        hr = jnp.maximum(sr_ * lax.rsqrt(dr) + b_ref[...], 0.0)
        xc = 0.5 * (hl + hr)
        degc = dl + dr - 1.0
        o_ref[...] = jnp.dot(xc, w_ref[...],
                             preferred_element_type=jnp.float32) * lax.rsqrt(degc)

    return pl.pallas_call(
        body,
        grid=(Rc // B,),
        in_specs=[
            pl.BlockSpec((1, B, H2), lambda i: (0, i, 0)),
            pl.BlockSpec((1, B, H2), lambda i: (1, i, 0)),
            pl.BlockSpec((B, H2), lambda i: (i, 0)),
            pl.BlockSpec((1, B, 16), lambda i: (0, i, 0)),
            pl.BlockSpec((1, B, 16), lambda i: (1, i, 0)),
            pl.BlockSpec((H, D), lambda i: (0, 0)),
            pl.BlockSpec((1, H), lambda i: (0, 0)),
        ],
        out_specs=pl.BlockSpec((B, D), lambda i: (i, 0)),
        out_shape=jax.ShapeDtypeStruct((Rc, D), jnp.float32),
    )(a0v, a1v, hsv, degv, degv, W, b)


def _tc_final(accc, hds, degv, b, B=640):
    Rc, D = hds.shape

    def body(a0_ref, a1_ref, hds_ref, d0_ref, d1_ref, b_ref, o_ref):
        dl = d0_ref[0, :, 0:1] + d1_ref[0, :, 0:1] + 1.0
        dr = d0_ref[0, :, 8:9] + d1_ref[0, :, 8:9] + 1.0
        degc = dl + dr - 1.0
        xd = ((a0_ref[0] + a1_ref[0] + hds_ref[...]) * lax.rsqrt(degc)
              + b_ref[...])
        o_ref[:, :D] = xd
        o_ref[:, D:] = xd

    return pl.pallas_call(
        body,
        grid=(Rc // B,),
        in_specs=[
            pl.BlockSpec((1, B, D), lambda i: (0, i, 0)),
            pl.BlockSpec((1, B, D), lambda i: (1, i, 0)),
            pl.BlockSpec((B, D), lambda i: (i, 0)),
            pl.BlockSpec((1, B, 16), lambda i: (0, i, 0)),
            pl.BlockSpec((1, B, 16), lambda i: (1, i, 0)),
            pl.BlockSpec((1, D), lambda i: (0, 0)),
        ],
        out_specs=pl.BlockSpec((B, 2 * D), lambda i: (i, 0)),
        out_shape=jax.ShapeDtypeStruct((Rc, 2 * D), jnp.float32),
    )(accc, accc, hds, degv, degv, b)


def kernel(x, edge_index, batch, W_enc, b_enc, W_dec, b_dec):
    N, D = x.shape
    H = W_enc.shape[1]
    E = edge_index.shape[1]
    Nc = N // 2

    Rc = ((Nc + 1 + 255) // 256) * 256
    R = 2 * Rc
    S = -(-(-(-E // CH)) // (NS * 8)) * 8
    cpt = -(-(-(-S // NC)) // 4) * 4
    iters = NS * S // NW
    C_pad = max(NS * S, NW * cpt)
    pad_e = C_pad * CH - E

    epk = edge_index[0] | (edge_index[1] << 16)
    ep = jnp.concatenate(
        [epk, jnp.full((pad_e,), N | (N << 16), jnp.int32)]).reshape(-1, CH)

    zeros8 = jnp.zeros((R, 8), jnp.float32)
    ones8 = jnp.zeros((CH, 8), jnp.float32).at[:, 0].set(1.0)
    zf = jnp.zeros((CH, H), jnp.float32)
    zc = jnp.zeros((CH, D), jnp.float32)
    x_pad = jnp.concatenate([x, jnp.zeros((R - N, D), x.dtype)])

    degp = _sc_degree(ep, zeros8, ones8, R, iters)
    hs = _tc_prep_enc(x_pad, W_enc, degp)
    accf = _sc_edge_pass(ep, hs, zf, R, H, False, cpt, 4)

    degv = degp.reshape(NC, Rc, 16)
    hds = _tc_mid(accf.reshape(NC, Rc, 2 * H), accf.reshape(NC, Rc, 2 * H),
                  hs.reshape(Rc, 2 * H), degv, W_dec, b_enc.reshape(1, H))
    accc = _sc_edge_pass(ep, hds, zc, Rc, D, True, cpt, 2)
    outd = _tc_final(accc, hds, degv, b_dec.reshape(1, D))

    return outd[:Nc].reshape(N, D)


# speedup vs baseline: 1.0753x; 1.0008x over previous
import functools

import jax
import jax.numpy as jnp
from jax import lax
from jax.experimental import pallas as pl
from jax.experimental.pallas import tpu as pltpu
from jax.experimental.pallas import tpu_sc as plsc

NC = 2
NS = 16
NW = NC * NS
CH = 128

_SC_PARAMS = pltpu.CompilerParams(use_tc_tiling_on_sc=False)


def _sc_degree(ep, zeros8, ones8, R, iters):
    rpt = R // NS
    mesh = plsc.VectorSubcoreMesh(core_axis_name="c", subcore_axis_name="s")
    K = 8
    rounds = iters // K

    @functools.partial(
        pl.kernel,
        out_type=jax.ShapeDtypeStruct((NC, R, 8), jnp.float32),
        mesh=mesh,
        scratch_types=[
            pltpu.VMEM((iters, CH), jnp.int32),
            pltpu.VMEM((K, CH), jnp.int32),
            pltpu.VMEM((CH, 8), jnp.float32),
            pltpu.VMEM((rpt, 8), jnp.float32),
            pltpu.VMEM_SHARED((R, 8), jnp.float32),
            pltpu.SemaphoreType.DMA,
        ],
        compiler_params=_SC_PARAMS,
    )
    def k(ep_hbm, zeros_hbm, ones_hbm, out_hbm, ep_v, idxd32, ones_v,
          chunk_v, hist, sem):
        cid = lax.axis_index("c")
        sid = lax.axis_index("s")
        wid = sid * NC + cid
        row = pl.ds(sid * rpt, rpt)
        pltpu.sync_copy(zeros_hbm.at[row], chunk_v)
        pltpu.sync_copy(chunk_v, hist.at[row])
        pltpu.sync_copy(ep_hbm.at[pl.ds(wid * iters, iters)], ep_v)
        pltpu.sync_copy(ones_hbm, ones_v)
        plsc.subcore_barrier()

        def body(g, carry):
            for b in range(K):
                for j in range(CH // 16):
                    v = ep_v[g * K + b, pl.ds(j * 16, 16)]
                    idxd32[b, pl.ds(j * 16, 16)] = v >> 16
                pltpu.async_copy(
                    ones_v, hist.at[idxd32.at[b]], sem, add=True)
            for b in range(K):
                pltpu.make_async_copy(
                    ones_v, hist.at[idxd32.at[0]], sem).wait()
            return carry

        lax.fori_loop(0, rounds, body, 0)
        plsc.subcore_barrier()
        pltpu.sync_copy(hist.at[row], chunk_v)
        pltpu.sync_copy(chunk_v, out_hbm.at[cid, row])

    return k(ep, zeros8, ones8)


def _sc_edge_pass(ep, table, zeros, R, W, shift, cpt, nb):
    rpt = R // NS
    mesh = plsc.VectorSubcoreMesh(core_axis_name="c", subcore_axis_name="s")
    NB = nb

    @functools.partial(
        pl.kernel,
        out_type=jax.ShapeDtypeStruct((NC, R, W), jnp.float32),
        mesh=mesh,
        scratch_types=(
            [pltpu.VMEM((cpt, CH), jnp.int32),
             pltpu.VMEM((NB, CH), jnp.int32),
             pltpu.VMEM((NB, CH), jnp.int32)]
            + [pltpu.VMEM((CH, W), jnp.float32) for _ in range(NB)]
            + [pltpu.VMEM_SHARED((R, W), jnp.float32),
               pltpu.VMEM_SHARED((R, W), jnp.float32)]
            + [pltpu.SemaphoreType.DMA for _ in range(2 * NB)]
        ),
        compiler_params=_SC_PARAMS,
    )
    def k(ep_hbm, table_hbm, zeros_hbm, out_hbm,
          ep_v, idxs32, idxd32, *bufs_and_sems):
        rows = bufs_and_sems[:NB]
        acc = bufs_and_sems[NB]
        table_sh = bufs_and_sems[NB + 1]
        semg = bufs_and_sems[NB + 2:NB + 2 + NB]
        sems = bufs_and_sems[NB + 2 + NB:]
        cid = lax.axis_index("c")
        sid = lax.axis_index("s")
        wid = sid * NC + cid
        row = pl.ds(sid * rpt, rpt)

        chunks = []
        o = 0
        while o < rpt:
            c = min(CH, rpt - o)
            chunks.append((o, c))
            o += c
        pltpu.sync_copy(table_hbm.at[row], table_sh.at[row])
        pltpu.sync_copy(zeros_hbm, rows[0])
        for (o, c) in chunks:
            pltpu.sync_copy(rows[0].at[pl.ds(0, c)],
                            acc.at[pl.ds(sid * rpt + o, c)])
        pltpu.sync_copy(ep_hbm.at[pl.ds(wid * cpt, cpt)], ep_v)
        plsc.subcore_barrier()

        def widen(b, i):
            for j in range(CH // 16):
                v = ep_v[i, pl.ds(j * 16, 16)]
                lo = v & 0xFFFF
                hi = v >> 16
                if shift:
                    lo = lo >> 1
                    hi = hi >> 1
                idxs32[b, pl.ds(j * 16, 16)] = lo
                idxd32[b, pl.ds(j * 16, 16)] = hi

        def body(g, carry):
            for b in range(NB):
                @pl.when(g > 0)
                def _drain():
                    pltpu.make_async_copy(
                        rows[b], acc.at[idxd32.at[0]], sems[b]).wait()
                widen(b, g * NB + b)
                pltpu.async_copy(
                    table_sh.at[idxs32.at[b]], rows[b], semg[b])
            for b in range(NB):
                pltpu.make_async_copy(
                    table_sh.at[idxs32.at[0]], rows[b], semg[b]).wait()
                pltpu.async_copy(
                    rows[b], acc.at[idxd32.at[b]], sems[b], add=True)
            return carry

        lax.fori_loop(0, cpt // NB, body, 0)
        for b in range(NB):
            pltpu.make_async_copy(
                rows[b], acc.at[idxd32.at[0]], sems[b]).wait()
        plsc.subcore_barrier()
        live = {}
        for z, (o, c) in enumerate(chunks):
            sl = pl.ds(sid * rpt + o, c)
            b = z % NB
            if b in live:
                pltpu.make_async_copy(
                    rows[b].at[pl.ds(0, live[b])],
                    out_hbm.at[cid, pl.ds(0, live[b])], semg[b]).wait()
            pltpu.sync_copy(acc.at[sl], rows[b].at[pl.ds(0, c)])
            pltpu.async_copy(rows[b].at[pl.ds(0, c)],
                             out_hbm.at[cid, sl], semg[b])
            live[b] = c
        for b, c in live.items():
            pltpu.make_async_copy(
                rows[b].at[pl.ds(0, c)],
                out_hbm.at[cid, pl.ds(0, c)], semg[b]).wait()

    return k(ep, table, zeros)


def _tc_prep_enc(x_pad, W, degp, B=640):
    R, D = x_pad.shape
    H = W.shape[1]

    def body(x_ref, w_ref, d_ref, o_ref):
        dinv = lax.rsqrt(d_ref[0, :, 0:1] + d_ref[1, :, 0:1] + 1.0)
        o_ref[...] = jnp.dot(x_ref[...], w_ref[...],
                             preferred_element_type=jnp.float32) * dinv

    return pl.pallas_call(
        body,
        grid=(R // B,),
        in_specs=[
            pl.BlockSpec((B, D), lambda i: (i, 0)),
            pl.BlockSpec((D, H), lambda i: (0, 0)),
            pl.BlockSpec((2, B, 8), lambda i: (0, i, 0)),
        ],
        out_specs=pl.BlockSpec((B, H), lambda i: (i, 0)),
        out_shape=jax.ShapeDtypeStruct((R, H), jnp.float32),
    )(x_pad, W, degp)


def _tc_mid(a0v, hsv, degv, W, b, B=640):
    Rc, H2 = hsv.shape
    H = H2 // 2
    D = W.shape[1]

    def body(a_ref, hs_ref, d_ref, w_ref, b_ref, o_ref):
        dl = d_ref[0, :, 0:1] + d_ref[1, :, 0:1] + 1.0
        dr = d_ref[0, :, 8:9] + d_ref[1, :, 8:9] + 1.0
        sl_ = (a_ref[0, :, :H] + a_ref[1, :, :H] + hs_ref[:, :H])
        sr_ = (a_ref[0, :, H:] + a_ref[1, :, H:] + hs_ref[:, H:])
        hl = jnp.maximum(sl_ * lax.rsqrt(dl) + b_ref[...], 0.0)
        hr = jnp.maximum(sr_ * lax.rsqrt(dr) + b_ref[...], 0.0)
        xc = 0.5 * (hl + hr)
        degc = dl + dr - 1.0
        o_ref[...] = jnp.dot(xc, w_ref[...],
                             preferred_element_type=jnp.float32) * lax.rsqrt(degc)

    return pl.pallas_call(
        body,
        grid=(Rc // B,),
        in_specs=[
            pl.BlockSpec((2, B, H2), lambda i: (0, i, 0)),
            pl.BlockSpec((B, H2), lambda i: (i, 0)),
            pl.BlockSpec((2, B, 16), lambda i: (0, i, 0)),
            pl.BlockSpec((H, D), lambda i: (0, 0)),
            pl.BlockSpec((1, H), lambda i: (0, 0)),
        ],
        out_specs=pl.BlockSpec((B, D), lambda i: (i, 0)),
        out_shape=jax.ShapeDtypeStruct((Rc, D), jnp.float32),
    )(a0v, hsv, degv, W, b)


def _tc_final(accc, hds, degv, b, B=640):
    Rc, D = hds.shape

    def body(a_ref, hds_ref, d_ref, b_ref, o_ref):
        dl = d_ref[0, :, 0:1] + d_ref[1, :, 0:1] + 1.0
        dr = d_ref[0, :, 8:9] + d_ref[1, :, 8:9] + 1.0
        degc = dl + dr - 1.0
        xd = ((a_ref[0] + a_ref[1] + hds_ref[...]) * lax.rsqrt(degc)
              + b_ref[...])
        o_ref[:, :D] = xd
        o_ref[:, D:] = xd

    return pl.pallas_call(
        body,
        grid=(Rc // B,),
        in_specs=[
            pl.BlockSpec((2, B, D), lambda i: (0, i, 0)),
            pl.BlockSpec((B, D), lambda i: (i, 0)),
            pl.BlockSpec((2, B, 16), lambda i: (0, i, 0)),
            pl.BlockSpec((1, D), lambda i: (0, 0)),
        ],
        out_specs=pl.BlockSpec((B, 2 * D), lambda i: (i, 0)),
        out_shape=jax.ShapeDtypeStruct((Rc, 2 * D), jnp.float32),
    )(accc, hds, degv, b)


def kernel(x, edge_index, batch, W_enc, b_enc, W_dec, b_dec):
    N, D = x.shape
    H = W_enc.shape[1]
    E = edge_index.shape[1]
    Nc = N // 2

    Rc = ((Nc + 1 + 255) // 256) * 256
    R = 2 * Rc
    S = -(-(-(-E // CH)) // (NS * 8)) * 8
    cpt = -(-(-(-S // NC)) // 4) * 4
    iters = NS * S // NW
    C_pad = max(NS * S, NW * cpt)
    pad_e = C_pad * CH - E

    epk = edge_index[0] | (edge_index[1] << 16)
    ep = jnp.concatenate(
        [epk, jnp.full((pad_e,), N | (N << 16), jnp.int32)]).reshape(-1, CH)

    zeros8 = jnp.zeros((R, 8), jnp.float32)
    ones8 = jnp.zeros((CH, 8), jnp.float32).at[:, 0].set(1.0)
    zf = jnp.zeros((CH, H), jnp.float32)
    zc = jnp.zeros((CH, D), jnp.float32)
    x_pad = jnp.concatenate([x, jnp.zeros((R - N, D), x.dtype)])

    degp = _sc_degree(ep, zeros8, ones8, R, iters)
    hs = _tc_prep_enc(x_pad, W_enc, degp)
    accf = _sc_edge_pass(ep, hs, zf, R, H, False, cpt, 4)

    degv = degp.reshape(NC, Rc, 16)
    hds = _tc_mid(accf.reshape(NC, Rc, 2 * H),
                  hs.reshape(Rc, 2 * H), degv, W_dec, b_enc.reshape(1, H))
    accc = _sc_edge_pass(ep, hds, zc, Rc, D, True, cpt, 2)
    outd = _tc_final(accc, hds, degv, b_dec.reshape(1, D))

    return outd[:Nc].reshape(N, D)
